# baseline (speedup 1.0000x reference)
import functools

import jax
import jax.numpy as jnp
from jax import lax
from jax.experimental import pallas as pl
from jax.experimental.pallas import tpu as pltpu
from jax.experimental.pallas import tpu_sc as plsc

NC = 2
NS = 16
NW = NC * NS


def _make_agg(n_pad, d, nchunks, k, with_counts):
  n = n_pad
  stripe = n // NS
  del with_counts
  mesh = plsc.VectorSubcoreMesh(core_axis_name="c", subcore_axis_name="s")
  out_type = [jax.ShapeDtypeStruct((NC, n, d), jnp.float32)]
  BK = 16
  assert nchunks % BK == 0 and BK % 2 == 0
  nbatch = nchunks // BK
  scratch = [
      pltpu.VMEM((BK, k), jnp.int32),
      pltpu.VMEM((BK, k), jnp.int32),
      pltpu.VMEM((k, d), jnp.float32),
      pltpu.VMEM((k, d), jnp.float32),
      pltpu.VMEM_SHARED((n, d), jnp.float32),
      pltpu.SemaphoreType.DMA,
      pltpu.SemaphoreType.DMA,
  ]

  def body(x_hbm, src_hbm, dst_hbm, sums_hbm,
           idx_s, idx_d, rows0, rows1, accum, sem0, sem1):
    rowsb = (rows0, rows1)
    semsb = (sem0, sem1)
    c = lax.axis_index("c")
    s = lax.axis_index("s")
    base = s * stripe

    zv = jnp.zeros((16,), jnp.float32)

    def zrow_step(i, carry):
      for cc in range(d // 16):
        rows0[i, pl.ds(cc * 16, 16)] = zv
      return carry

    lax.fori_loop(0, 8, zrow_step, 0)
    assert stripe % 8 == 0

    def zcopy_step(t, carry):
      pltpu.sync_copy(rows0.at[pl.ds(0, 8)],
                      accum.at[pl.ds(base + t * 8, 8)])
      return carry

    lax.fori_loop(0, stripe // 8, zcopy_step, 0)
    plsc.subcore_barrier()

    def batch_step(bt, carry):
      pltpu.sync_copy(src_hbm.at[c, s, pl.ds(bt * BK, BK)], idx_s)
      pltpu.sync_copy(dst_hbm.at[c, s, pl.ds(bt * BK, BK)], idx_d)
      pltpu.async_copy(x_hbm.at[idx_s.at[0]], rows0, sem0)

      def pair_step(j2, carry2):
        for b in range(2):
          j = j2 * 2 + b
          pltpu.async_copy(x_hbm.at[idx_s.at[j + 1]], rowsb[1 - b],
                           semsb[1 - b])
          pltpu.make_async_copy(x_hbm.at[idx_s.at[j]], rowsb[b],
                                semsb[b]).wait()
          pltpu.sync_copy(rowsb[b], accum.at[idx_d.at[j]], add=True)
        return carry2

      lax.fori_loop(0, BK // 2 - 1, pair_step, 0)
      jt = BK - 2
      pltpu.async_copy(x_hbm.at[idx_s.at[jt + 1]], rows1, sem1)
      pltpu.make_async_copy(x_hbm.at[idx_s.at[jt]], rows0, sem0).wait()
      pltpu.sync_copy(rows0, accum.at[idx_d.at[jt]], add=True)
      pltpu.make_async_copy(x_hbm.at[idx_s.at[jt + 1]], rows1, sem1).wait()
      pltpu.sync_copy(rows1, accum.at[idx_d.at[jt + 1]], add=True)
      return carry

    lax.fori_loop(0, nbatch, batch_step, 0)
    plsc.subcore_barrier()
    pltpu.sync_copy(accum.at[pl.ds(base, stripe)],
                    sums_hbm.at[c, pl.ds(base, stripe)])

  return pl.kernel(
      body, out_type=out_type, mesh=mesh, scratch_types=scratch,
      compiler_params=pltpu.CompilerParams(use_tc_tiling_on_sc=False))


def _make_counts(n_pad, ngroups):
  mesh = plsc.VectorSubcoreMesh(core_axis_name="c", subcore_axis_name="s")
  nr = n_pad // 128
  out_type = [jax.ShapeDtypeStruct((NC, NS, nr, 128), jnp.float32)]
  scratch = [
      pltpu.VMEM((ngroups, 16), jnp.int32),
      pltpu.VMEM((nr, 128), jnp.float32),
  ]

  def body(dst_hbm, zn_hbm, cnt_hbm, idx_d, hist):
    c = lax.axis_index("c")
    s = lax.axis_index("s")
    pltpu.sync_copy(zn_hbm, hist)
    pltpu.sync_copy(dst_hbm.at[c, s], idx_d)
    ones = jnp.full((16,), 1.0, jnp.float32)

    def step(j, carry):
      idx = idx_d[j]
      row = lax.shift_right_logical(idx, 7)
      col = lax.bitwise_and(idx, 127)
      plsc.addupdate_scatter(hist, [row, col], ones)
      return carry

    lax.fori_loop(0, ngroups, step, 0)
    pltpu.sync_copy(hist, cnt_hbm.at[c, s])

  return pl.kernel(
      body, out_type=out_type, mesh=mesh, scratch_types=scratch,
      compiler_params=pltpu.CompilerParams(needs_layout_passes=False))


def _dotT(a, w):
  return lax.dot_general(a, w, (((1,), (1,)), ((), ())),
                         preferred_element_type=jnp.float32)


def _tc1_body(x_ref, s_ref, c_ref, wl_ref, wr_ref, o_ref):
  cnt = jnp.sum(c_ref[...], axis=(0, 1))[:, None]
  agg = (s_ref[0] + s_ref[1]) / jnp.maximum(cnt, 1.0)
  out = _dotT(x_ref[...], wl_ref[...]) + _dotT(agg, wr_ref[...])
  nrm = jnp.sqrt(jnp.sum(out * out, axis=1, keepdims=True))
  out = out / jnp.maximum(nrm, 1e-12)
  o_ref[...] = jnp.maximum(out, 0.0)


def _tc2_body(x_ref, s_ref, c_ref, wl_ref, wr_ref,
              wp1_ref, bp1_ref, wp2_ref, bp2_ref, o_ref):
  cnt = jnp.sum(c_ref[...], axis=(0, 1))[:, None]
  agg = (s_ref[0] + s_ref[1]) / jnp.maximum(cnt, 1.0)
  out = _dotT(x_ref[...], wl_ref[...]) + _dotT(agg, wr_ref[...])
  nrm = jnp.sqrt(jnp.sum(out * out, axis=1, keepdims=True))
  out = out / jnp.maximum(nrm, 1e-12)
  out = jnp.maximum(out, 0.0)
  out = _dotT(out, wp1_ref[...]) + bp1_ref[...]
  out = _dotT(out, wp2_ref[...]) + bp2_ref[...]
  o_ref[...] = out


def kernel(x, edge_index, batch, W_l1, W_r1, W_l2, W_r2, W_p1, b_p1, W_p2,
           b_p2):
  n, d = x.shape
  e = edge_index.shape[1]
  k = 128
  nchunks = -(-e // (NW * k))
  nchunks += (-nchunks) % 16
  ew = nchunks * k

  e_pad = ew * NW
  n_pad = -(-n // (NS * 8)) * (NS * 8)
  if e_pad > e and n_pad == n:
    n_pad += NS * 8
  stripe = n_pad // NS

  src_flat = jnp.concatenate(
      [edge_index[0], jnp.zeros((e_pad - e,), jnp.int32)])
  dst_flat = jnp.concatenate(
      [edge_index[1], jnp.full((e_pad - e,), n, jnp.int32)])
  src_r = src_flat.reshape(NC, NS, nchunks, k)
  dst_r = dst_flat.reshape(NC, NS, nchunks, k)
  dst_g = edge_index[1].reshape(NC, NS, (e // NW) // 16, 16)
  zn = jnp.zeros((n_pad // 128, 128), jnp.float32)

  (cnt,) = _make_counts(n_pad, (e // NW) // 16)(dst_g, zn)
  cnt = cnt.reshape(NC, NS, n_pad)
  agg_a = _make_agg(n_pad, d, nchunks, k, False)
  (sums1,) = agg_a(x, src_r, dst_r)

  x_p = jnp.pad(x, ((0, n_pad - n), (0, 0)))
  bn = 128
  grid = (n_pad // bn,)
  row_spec = pl.BlockSpec((bn, d), lambda i: (i, 0))
  sum_spec = pl.BlockSpec((NC, bn, d), lambda i: (0, i, 0))
  cnt_spec = pl.BlockSpec((NC, NS, bn), lambda i: (0, 0, i))
  w_spec = pl.BlockSpec((d, d), lambda i: (0, 0))
  b_spec = pl.BlockSpec((1, d), lambda i: (0, 0))

  h1 = pl.pallas_call(
      _tc1_body,
      grid=grid,
      in_specs=[row_spec, sum_spec, cnt_spec, w_spec, w_spec],
      out_specs=row_spec,
      out_shape=jax.ShapeDtypeStruct((n_pad, d), jnp.float32),
  )(x_p, sums1, cnt, W_l1, W_r1)

  agg_b = _make_agg(n_pad, d, nchunks, k, False)
  (sums2,) = agg_b(h1, src_r, dst_r)

  out = pl.pallas_call(
      _tc2_body,
      grid=grid,
      in_specs=[row_spec, sum_spec, cnt_spec, w_spec, w_spec,
                w_spec, b_spec, w_spec, b_spec],
      out_specs=row_spec,
      out_shape=jax.ShapeDtypeStruct((n_pad, d), jnp.float32),
  )(h1, sums2, cnt, W_l2, W_r2, W_p1, b_p1.reshape(1, d), W_p2,
    b_p2.reshape(1, d))
  return out[:n]


# speedup vs baseline: 1.0678x; 1.0678x over previous
import functools

import jax
import jax.numpy as jnp
from jax import lax
from jax.experimental import pallas as pl
from jax.experimental.pallas import tpu as pltpu
from jax.experimental.pallas import tpu_sc as plsc

NC = 2
NS = 16
NW = NC * NS


def _make_agg(n_pad, d, nchunks, k, with_counts):
  n = n_pad
  stripe = n // NS
  del with_counts
  mesh = plsc.VectorSubcoreMesh(core_axis_name="c", subcore_axis_name="s")
  out_type = [jax.ShapeDtypeStruct((NC, n, d), jnp.float32)]
  BK = 16
  assert nchunks % BK == 0 and BK % 2 == 0
  nbatch = nchunks // BK
  scratch = [
      pltpu.VMEM((BK, k), jnp.int32),
      pltpu.VMEM((BK, k), jnp.int32),
      pltpu.VMEM((k, d), jnp.float32),
      pltpu.VMEM((k, d), jnp.float32),
      pltpu.VMEM_SHARED((n, d), jnp.float32),
      pltpu.SemaphoreType.DMA,
      pltpu.SemaphoreType.DMA,
  ]

  def body(x_hbm, src_hbm, dst_hbm, sums_hbm,
           idx_s, idx_d, rows0, rows1, accum, sem0, sem1):
    rowsb = (rows0, rows1)
    semsb = (sem0, sem1)
    c = lax.axis_index("c")
    s = lax.axis_index("s")
    base = s * stripe

    zv = jnp.zeros((16,), jnp.float32)

    def zrow_step(i, carry):
      for cc in range(d // 16):
        rows0[i, pl.ds(cc * 16, 16)] = zv
      return carry

    lax.fori_loop(0, 8, zrow_step, 0)
    assert stripe % 8 == 0

    def zcopy_step(t, carry):
      pltpu.sync_copy(rows0.at[pl.ds(0, 8)],
                      accum.at[pl.ds(base + t * 8, 8)])
      return carry

    lax.fori_loop(0, stripe // 8, zcopy_step, 0)
    plsc.subcore_barrier()

    def batch_step(bt, carry):
      pltpu.sync_copy(src_hbm.at[c, s, pl.ds(bt * BK, BK)], idx_s)
      pltpu.sync_copy(dst_hbm.at[c, s, pl.ds(bt * BK, BK)], idx_d)
      pltpu.async_copy(x_hbm.at[idx_s.at[0]], rows0, sem0)

      def pair_step(j2, carry2):
        for b in range(2):
          j = j2 * 2 + b
          pltpu.async_copy(x_hbm.at[idx_s.at[j + 1]], rowsb[1 - b],
                           semsb[1 - b])
          pltpu.make_async_copy(x_hbm.at[idx_s.at[j]], rowsb[b],
                                semsb[b]).wait()
          pltpu.sync_copy(rowsb[b], accum.at[idx_d.at[j]], add=True)
        return carry2

      lax.fori_loop(0, BK // 2 - 1, pair_step, 0)
      jt = BK - 2
      pltpu.async_copy(x_hbm.at[idx_s.at[jt + 1]], rows1, sem1)
      pltpu.make_async_copy(x_hbm.at[idx_s.at[jt]], rows0, sem0).wait()
      pltpu.sync_copy(rows0, accum.at[idx_d.at[jt]], add=True)
      pltpu.make_async_copy(x_hbm.at[idx_s.at[jt + 1]], rows1, sem1).wait()
      pltpu.sync_copy(rows1, accum.at[idx_d.at[jt + 1]], add=True)
      return carry

    lax.fori_loop(0, nbatch, batch_step, 0)
    plsc.subcore_barrier()
    pltpu.sync_copy(accum.at[pl.ds(base, stripe)],
                    sums_hbm.at[c, pl.ds(base, stripe)])

  return pl.kernel(
      body, out_type=out_type, mesh=mesh, scratch_types=scratch,
      compiler_params=pltpu.CompilerParams(use_tc_tiling_on_sc=False))


def _make_counts(n_pad, ngroups):
  mesh = plsc.VectorSubcoreMesh(core_axis_name="c", subcore_axis_name="s")
  nr = n_pad // 128
  out_type = [jax.ShapeDtypeStruct((NC, NS, nr, 128), jnp.float32)]
  scratch = [
      pltpu.VMEM((ngroups, 16), jnp.int32),
      pltpu.VMEM((nr, 128), jnp.float32),
  ]

  def body(dst_hbm, zn_hbm, cnt_hbm, idx_d, hist):
    c = lax.axis_index("c")
    s = lax.axis_index("s")
    pltpu.sync_copy(zn_hbm, hist)
    pltpu.sync_copy(dst_hbm.at[c, s], idx_d)
    ones = jnp.full((16,), 1.0, jnp.float32)

    def step(j, carry):
      idx = idx_d[j]
      row = lax.shift_right_logical(idx, 7)
      col = lax.bitwise_and(idx, 127)
      plsc.addupdate_scatter(hist, [row, col], ones)
      return carry

    lax.fori_loop(0, ngroups, step, 0)
    pltpu.sync_copy(hist, cnt_hbm.at[c, s])

  return pl.kernel(
      body, out_type=out_type, mesh=mesh, scratch_types=scratch,
      compiler_params=pltpu.CompilerParams(needs_layout_passes=False))


def _dotT(a, w):
  return lax.dot_general(a, w, (((1,), (1,)), ((), ())),
                         preferred_element_type=jnp.float32)


def _tc1_body(x_ref, s_ref, c_ref, wl_ref, wr_ref, o_ref):
  cnt = jnp.sum(c_ref[...], axis=(0, 1))[:, None]
  agg = (s_ref[0] + s_ref[1]) / jnp.maximum(cnt, 1.0)
  out = _dotT(x_ref[...], wl_ref[...]) + _dotT(agg, wr_ref[...])
  nrm = jnp.sqrt(jnp.sum(out * out, axis=1, keepdims=True))
  out = out / jnp.maximum(nrm, 1e-12)
  o_ref[...] = jnp.maximum(out, 0.0)


def _tc2_body(x_ref, s_ref, c_ref, wl_ref, wr_ref,
              wp1_ref, bp1_ref, wp2_ref, bp2_ref, o_ref):
  cnt = jnp.sum(c_ref[...], axis=(0, 1))[:, None]
  agg = (s_ref[0] + s_ref[1]) / jnp.maximum(cnt, 1.0)
  out = _dotT(x_ref[...], wl_ref[...]) + _dotT(agg, wr_ref[...])
  nrm = jnp.sqrt(jnp.sum(out * out, axis=1, keepdims=True))
  out = out / jnp.maximum(nrm, 1e-12)
  out = jnp.maximum(out, 0.0)
  out = _dotT(out, wp1_ref[...]) + bp1_ref[...]
  out = _dotT(out, wp2_ref[...]) + bp2_ref[...]
  o_ref[...] = out


def kernel(x, edge_index, batch, W_l1, W_r1, W_l2, W_r2, W_p1, b_p1, W_p2,
           b_p2):
  n, d = x.shape
  e = edge_index.shape[1]
  k = 128
  nchunks = -(-e // (NW * k))
  nchunks += (-nchunks) % 16
  ew = nchunks * k

  e_pad = ew * NW
  n_pad = -(-n // (NS * 8)) * (NS * 8)
  if e_pad > e and n_pad == n:
    n_pad += NS * 8
  stripe = n_pad // NS

  ew_real = e // NW
  pad_w = ew - ew_real
  src_w = edge_index[0].reshape(NW, ew_real)
  dst_w = edge_index[1].reshape(NW, ew_real)
  if pad_w:
    dummy_dst = n + (jnp.arange(pad_w, dtype=jnp.int32) % (n_pad - n))
    src_w = jnp.pad(src_w, ((0, 0), (0, pad_w)))
    dst_w = jnp.concatenate(
        [dst_w, jnp.broadcast_to(dummy_dst, (NW, pad_w))], axis=1)
  src_r = src_w.reshape(NC, NS, nchunks, k)
  dst_r = dst_w.reshape(NC, NS, nchunks, k)
  dst_g = edge_index[1].reshape(NC, NS, (e // NW) // 16, 16)
  zn = jnp.zeros((n_pad // 128, 128), jnp.float32)

  (cnt,) = _make_counts(n_pad, (e // NW) // 16)(dst_g, zn)
  cnt = cnt.reshape(NC, NS, n_pad)
  agg_a = _make_agg(n_pad, d, nchunks, k, False)
  (sums1,) = agg_a(x, src_r, dst_r)

  x_p = jnp.pad(x, ((0, n_pad - n), (0, 0)))
  bn = 128
  grid = (n_pad // bn,)
  row_spec = pl.BlockSpec((bn, d), lambda i: (i, 0))
  sum_spec = pl.BlockSpec((NC, bn, d), lambda i: (0, i, 0))
  cnt_spec = pl.BlockSpec((NC, NS, bn), lambda i: (0, 0, i))
  w_spec = pl.BlockSpec((d, d), lambda i: (0, 0))
  b_spec = pl.BlockSpec((1, d), lambda i: (0, 0))

  h1 = pl.pallas_call(
      _tc1_body,
      grid=grid,
      in_specs=[row_spec, sum_spec, cnt_spec, w_spec, w_spec],
      out_specs=row_spec,
      out_shape=jax.ShapeDtypeStruct((n_pad, d), jnp.float32),
  )(x_p, sums1, cnt, W_l1, W_r1)

  agg_b = _make_agg(n_pad, d, nchunks, k, False)
  (sums2,) = agg_b(h1, src_r, dst_r)

  out = pl.pallas_call(
      _tc2_body,
      grid=grid,
      in_specs=[row_spec, sum_spec, cnt_spec, w_spec, w_spec,
                w_spec, b_spec, w_spec, b_spec],
      out_specs=row_spec,
      out_shape=jax.ShapeDtypeStruct((n_pad, d), jnp.float32),
  )(h1, sums2, cnt, W_l2, W_r2, W_p1, b_p1.reshape(1, d), W_p2,
    b_p2.reshape(1, d))
  return out[:n]


# speedup vs baseline: 2.5094x; 2.3500x over previous
import functools

import jax
import jax.numpy as jnp
from jax import lax
from jax.experimental import pallas as pl
from jax.experimental.pallas import tpu as pltpu
from jax.experimental.pallas import tpu_sc as plsc

NC = 2
NS = 16
NW = NC * NS


def _make_agg(n_pad, d, nchunks, k, with_counts):
  n = n_pad
  stripe = n // NS
  del with_counts
  mesh = plsc.VectorSubcoreMesh(core_axis_name="c", subcore_axis_name="s")
  out_type = [jax.ShapeDtypeStruct((NC, n, d), jnp.float32)]
  BK = 16
  assert nchunks % BK == 0 and BK % 2 == 0
  nbatch = nchunks // BK
  scratch = [
      pltpu.VMEM((BK, k), jnp.int32),
      pltpu.VMEM((BK, k), jnp.int32),
      pltpu.VMEM((k, d), jnp.float32),
      pltpu.VMEM((k, d), jnp.float32),
      pltpu.VMEM_SHARED((n, d), jnp.float32),
      pltpu.SemaphoreType.DMA,
      pltpu.SemaphoreType.DMA,
  ]

  def body(x_hbm, src_hbm, dst_hbm, sums_hbm,
           idx_s, idx_d, rows0, rows1, accum, sem0, sem1):
    rowsb = (rows0, rows1)
    semsb = (sem0, sem1)
    c = lax.axis_index("c")
    s = lax.axis_index("s")
    base = s * stripe

    zv = jnp.zeros((16,), jnp.float32)

    def zrow_step(i, carry):
      for cc in range(d // 16):
        rows0[i, pl.ds(cc * 16, 16)] = zv
      return carry

    lax.fori_loop(0, 8, zrow_step, 0)
    assert stripe % 8 == 0

    def zcopy_step(t, carry):
      pltpu.sync_copy(rows0.at[pl.ds(0, 8)],
                      accum.at[pl.ds(base + t * 8, 8)])
      return carry

    lax.fori_loop(0, stripe // 8, zcopy_step, 0)
    plsc.subcore_barrier()

    def batch_step(bt, carry):
      pltpu.sync_copy(src_hbm.at[c, s, pl.ds(bt * BK, BK)], idx_s)
      pltpu.sync_copy(dst_hbm.at[c, s, pl.ds(bt * BK, BK)], idx_d)
      pltpu.async_copy(x_hbm.at[idx_s.at[0]], rows0, sem0)

      def pair_step(j2, carry2):
        for b in range(2):
          j = j2 * 2 + b
          pltpu.async_copy(x_hbm.at[idx_s.at[j + 1]], rowsb[1 - b],
                           semsb[1 - b])
          pltpu.make_async_copy(x_hbm.at[idx_s.at[j]], rowsb[b],
                                semsb[b]).wait()
          pltpu.sync_copy(rowsb[b], accum.at[idx_d.at[j]], add=True)
        return carry2

      lax.fori_loop(0, BK // 2 - 1, pair_step, 0)
      jt = BK - 2
      pltpu.async_copy(x_hbm.at[idx_s.at[jt + 1]], rows1, sem1)
      pltpu.make_async_copy(x_hbm.at[idx_s.at[jt]], rows0, sem0).wait()
      pltpu.sync_copy(rows0, accum.at[idx_d.at[jt]], add=True)
      pltpu.make_async_copy(x_hbm.at[idx_s.at[jt + 1]], rows1, sem1).wait()
      pltpu.sync_copy(rows1, accum.at[idx_d.at[jt + 1]], add=True)
      return carry

    lax.fori_loop(0, nbatch, batch_step, 0)
    plsc.subcore_barrier()
    pltpu.sync_copy(accum.at[pl.ds(base, stripe)],
                    sums_hbm.at[c, pl.ds(base, stripe)])

  return pl.kernel(
      body, out_type=out_type, mesh=mesh, scratch_types=scratch,
      compiler_params=pltpu.CompilerParams(use_tc_tiling_on_sc=False))


def _make_counts(n_pad, ngroups):
  mesh = plsc.VectorSubcoreMesh(core_axis_name="c", subcore_axis_name="s")
  nr = n_pad // 128
  out_type = [jax.ShapeDtypeStruct((NC, NS, nr, 128), jnp.float32)]
  scratch = [
      pltpu.VMEM((ngroups, 16), jnp.int32),
      pltpu.VMEM((nr, 128), jnp.float32),
  ]

  def body(dst_hbm, zn_hbm, cnt_hbm, idx_d, hist):
    c = lax.axis_index("c")
    s = lax.axis_index("s")
    pltpu.sync_copy(zn_hbm, hist)
    pltpu.sync_copy(dst_hbm.at[c, s], idx_d)
    ones = jnp.full((16,), 1.0, jnp.float32)

    def step(j, carry):
      idx = idx_d[j]
      row = lax.shift_right_logical(idx, 7)
      col = lax.bitwise_and(idx, 127)
      plsc.addupdate_scatter(hist, [row, col], ones)
      return carry

    lax.fori_loop(0, ngroups, step, 0)
    pltpu.sync_copy(hist, cnt_hbm.at[c, s])

  return pl.kernel(
      body, out_type=out_type, mesh=mesh, scratch_types=scratch,
      compiler_params=pltpu.CompilerParams(needs_layout_passes=False))


def _dotT(a, w):
  return lax.dot_general(a, w, (((1,), (1,)), ((), ())),
                         preferred_element_type=jnp.float32)


def _tc1_body(x_ref, s_ref, c_ref, wl_ref, wr_ref, o_ref):
  cnt = jnp.sum(c_ref[...], axis=(0, 1))[:, None]
  agg = (s_ref[0] + s_ref[1]) / jnp.maximum(cnt, 1.0)
  out = _dotT(x_ref[...], wl_ref[...]) + _dotT(agg, wr_ref[...])
  nrm = jnp.sqrt(jnp.sum(out * out, axis=1, keepdims=True))
  out = out / jnp.maximum(nrm, 1e-12)
  o_ref[...] = jnp.maximum(out, 0.0)


def _tc2_body(x_ref, s_ref, c_ref, wl_ref, wr_ref,
              wp1_ref, bp1_ref, wp2_ref, bp2_ref, o_ref):
  cnt = jnp.sum(c_ref[...], axis=(0, 1))[:, None]
  agg = (s_ref[0] + s_ref[1]) / jnp.maximum(cnt, 1.0)
  out = _dotT(x_ref[...], wl_ref[...]) + _dotT(agg, wr_ref[...])
  nrm = jnp.sqrt(jnp.sum(out * out, axis=1, keepdims=True))
  out = out / jnp.maximum(nrm, 1e-12)
  out = jnp.maximum(out, 0.0)
  out = _dotT(out, wp1_ref[...]) + bp1_ref[...]
  out = _dotT(out, wp2_ref[...]) + bp2_ref[...]
  o_ref[...] = out


def kernel(x, edge_index, batch, W_l1, W_r1, W_l2, W_r2, W_p1, b_p1, W_p2,
           b_p2):
  n, d = x.shape
  e = edge_index.shape[1]
  k = 128
  nchunks = -(-e // (NW * k))
  nchunks += (-nchunks) % 16
  ew = nchunks * k

  e_pad = ew * NW
  n_pad = -(-n // (NS * 8)) * (NS * 8)
  if e_pad > e and n_pad == n:
    n_pad += NS * 8
  stripe = n_pad // NS

  ew_real = e // NW
  pad_w = ew - ew_real
  src_w = edge_index[0].reshape(NW, ew_real)
  dst_w = edge_index[1].reshape(NW, ew_real)
  if pad_w:
    ar = jnp.arange(pad_w, dtype=jnp.int32)
    wi = jnp.arange(NW, dtype=jnp.int32)[:, None]
    dummy_src = jnp.broadcast_to(n + (ar % (n_pad - n)), (NW, pad_w))
    dummy_dst = (ar[None, :] * 997 + wi * 131) % n
    src_w = jnp.concatenate([src_w, dummy_src], axis=1)
    dst_w = jnp.concatenate([dst_w, dummy_dst], axis=1)
  src_r = src_w.reshape(NC, NS, nchunks, k)
  dst_r = dst_w.reshape(NC, NS, nchunks, k)
  dst_g = edge_index[1].reshape(NC, NS, (e // NW) // 16, 16)
  zn = jnp.zeros((n_pad // 128, 128), jnp.float32)

  (cnt,) = _make_counts(n_pad, (e // NW) // 16)(dst_g, zn)
  cnt = cnt.reshape(NC, NS, n_pad)
  agg_a = _make_agg(n_pad, d, nchunks, k, False)
  x_p = jnp.pad(x, ((0, n_pad - n), (0, 0)))
  (sums1,) = agg_a(x_p, src_r, dst_r)
  bn = 128
  grid = (n_pad // bn,)
  row_spec = pl.BlockSpec((bn, d), lambda i: (i, 0))
  sum_spec = pl.BlockSpec((NC, bn, d), lambda i: (0, i, 0))
  cnt_spec = pl.BlockSpec((NC, NS, bn), lambda i: (0, 0, i))
  w_spec = pl.BlockSpec((d, d), lambda i: (0, 0))
  b_spec = pl.BlockSpec((1, d), lambda i: (0, 0))

  h1 = pl.pallas_call(
      _tc1_body,
      grid=grid,
      in_specs=[row_spec, sum_spec, cnt_spec, w_spec, w_spec],
      out_specs=row_spec,
      out_shape=jax.ShapeDtypeStruct((n_pad, d), jnp.float32),
  )(x_p, sums1, cnt, W_l1, W_r1)

  agg_b = _make_agg(n_pad, d, nchunks, k, False)
  (sums2,) = agg_b(h1, src_r, dst_r)

  out = pl.pallas_call(
      _tc2_body,
      grid=grid,
      in_specs=[row_spec, sum_spec, cnt_spec, w_spec, w_spec,
                w_spec, b_spec, w_spec, b_spec],
      out_specs=row_spec,
      out_shape=jax.ShapeDtypeStruct((n_pad, d), jnp.float32),
  )(h1, sums2, cnt, W_l2, W_r2, W_p1, b_p1.reshape(1, d), W_p2,
    b_p2.reshape(1, d))
  return out[:n]


# speedup vs baseline: 2.5262x; 1.0067x over previous
import functools

import jax
import jax.numpy as jnp
from jax import lax
from jax.experimental import pallas as pl
from jax.experimental.pallas import tpu as pltpu
from jax.experimental.pallas import tpu_sc as plsc

NC = 2
NS = 16
NW = NC * NS


def _make_agg(n_pad, d, nchunks, k, with_counts):
  n = n_pad
  stripe = n // NS
  del with_counts
  mesh = plsc.VectorSubcoreMesh(core_axis_name="c", subcore_axis_name="s")
  out_type = [jax.ShapeDtypeStruct((NC, n, d), jnp.float32)]
  BK = 16
  assert nchunks % BK == 0 and BK % 2 == 0
  nbatch = nchunks // BK
  scratch = [
      pltpu.VMEM((BK, k), jnp.int32),
      pltpu.VMEM((BK, k), jnp.int32),
      pltpu.VMEM((k, d), jnp.float32),
      pltpu.VMEM((k, d), jnp.float32),
      pltpu.VMEM_SHARED((n, d), jnp.float32),
      pltpu.SemaphoreType.DMA,
      pltpu.SemaphoreType.DMA,
  ]

  def body(x_hbm, src_hbm, dst_hbm, sums_hbm,
           idx_s, idx_d, rows0, rows1, accum, sem0, sem1):
    rowsb = (rows0, rows1)
    semsb = (sem0, sem1)
    c = lax.axis_index("c")
    s = lax.axis_index("s")
    base = s * stripe

    zv = jnp.zeros((16,), jnp.float32)

    def zrow_step(i, carry):
      for cc in range(d // 16):
        rows0[i, pl.ds(cc * 16, 16)] = zv
      return carry

    lax.fori_loop(0, 8, zrow_step, 0)
    assert stripe % 8 == 0

    def zcopy_step(t, carry):
      pltpu.sync_copy(rows0.at[pl.ds(0, 8)],
                      accum.at[pl.ds(base + t * 8, 8)])
      return carry

    lax.fori_loop(0, stripe // 8, zcopy_step, 0)
    plsc.subcore_barrier()

    def batch_step(bt, carry):
      pltpu.sync_copy(src_hbm.at[c, s, pl.ds(bt * BK, BK)], idx_s)
      pltpu.sync_copy(dst_hbm.at[c, s, pl.ds(bt * BK, BK)], idx_d)
      pltpu.async_copy(x_hbm.at[idx_s.at[0]], rows0, sem0)

      def pair_step(j2, carry2):
        for b in range(2):
          j = j2 * 2 + b
          pltpu.async_copy(x_hbm.at[idx_s.at[j + 1]], rowsb[1 - b],
                           semsb[1 - b])
          pltpu.make_async_copy(x_hbm.at[idx_s.at[j]], rowsb[b],
                                semsb[b]).wait()
          pltpu.sync_copy(rowsb[b], accum.at[idx_d.at[j]], add=True)
        return carry2

      lax.fori_loop(0, BK // 2 - 1, pair_step, 0)
      jt = BK - 2
      pltpu.async_copy(x_hbm.at[idx_s.at[jt + 1]], rows1, sem1)
      pltpu.make_async_copy(x_hbm.at[idx_s.at[jt]], rows0, sem0).wait()
      pltpu.sync_copy(rows0, accum.at[idx_d.at[jt]], add=True)
      pltpu.make_async_copy(x_hbm.at[idx_s.at[jt + 1]], rows1, sem1).wait()
      pltpu.sync_copy(rows1, accum.at[idx_d.at[jt + 1]], add=True)
      return carry

    lax.fori_loop(0, nbatch, batch_step, 0)
    plsc.subcore_barrier()
    pltpu.sync_copy(accum.at[pl.ds(base, stripe)],
                    sums_hbm.at[c, pl.ds(base, stripe)])

  return pl.kernel(
      body, out_type=out_type, mesh=mesh, scratch_types=scratch,
      compiler_params=pltpu.CompilerParams(use_tc_tiling_on_sc=False))


def _make_counts(n_pad, ngroups):
  mesh = plsc.VectorSubcoreMesh(core_axis_name="c", subcore_axis_name="s")
  nr = n_pad // 128
  out_type = [jax.ShapeDtypeStruct((NC, NS, nr, 128), jnp.float32)]
  scratch = [
      pltpu.VMEM((ngroups, 16), jnp.int32),
      pltpu.VMEM((nr, 128), jnp.float32),
  ]

  def body(dst_hbm, zn_hbm, cnt_hbm, idx_d, hist):
    c = lax.axis_index("c")
    s = lax.axis_index("s")
    pltpu.sync_copy(zn_hbm, hist)
    pltpu.sync_copy(dst_hbm.at[c, s], idx_d)
    ones = jnp.full((16,), 1.0, jnp.float32)

    def step(j, carry):
      idx = idx_d[j]
      row = lax.shift_right_logical(idx, 7)
      col = lax.bitwise_and(idx, 127)
      plsc.addupdate_scatter(hist, [row, col], ones)
      return carry

    lax.fori_loop(0, ngroups, step, 0)
    pltpu.sync_copy(hist, cnt_hbm.at[c, s])

  return pl.kernel(
      body, out_type=out_type, mesh=mesh, scratch_types=scratch,
      compiler_params=pltpu.CompilerParams(needs_layout_passes=False))


def _dotT(a, w):
  return lax.dot_general(a, w, (((1,), (1,)), ((), ())),
                         preferred_element_type=jnp.float32)


def _tc1_body(x_ref, s_ref, c_ref, wl_ref, wr_ref, o_ref):
  cnt = jnp.sum(c_ref[...], axis=(0, 1))[:, None]
  agg = (s_ref[0] + s_ref[1]) / jnp.maximum(cnt, 1.0)
  out = _dotT(x_ref[...], wl_ref[...]) + _dotT(agg, wr_ref[...])
  nrm = jnp.sqrt(jnp.sum(out * out, axis=1, keepdims=True))
  out = out / jnp.maximum(nrm, 1e-12)
  o_ref[...] = jnp.maximum(out, 0.0)


def _tc2_body(x_ref, s_ref, c_ref, wl_ref, wr_ref,
              wp1_ref, bp1_ref, wp2_ref, bp2_ref, o_ref):
  cnt = jnp.sum(c_ref[...], axis=(0, 1))[:, None]
  agg = (s_ref[0] + s_ref[1]) / jnp.maximum(cnt, 1.0)
  out = _dotT(x_ref[...], wl_ref[...]) + _dotT(agg, wr_ref[...])
  nrm = jnp.sqrt(jnp.sum(out * out, axis=1, keepdims=True))
  out = out / jnp.maximum(nrm, 1e-12)
  out = jnp.maximum(out, 0.0)
  out = _dotT(out, wp1_ref[...]) + bp1_ref[...]
  out = _dotT(out, wp2_ref[...]) + bp2_ref[...]
  o_ref[...] = out


def kernel(x, edge_index, batch, W_l1, W_r1, W_l2, W_r2, W_p1, b_p1, W_p2,
           b_p2):
  n, d = x.shape
  e = edge_index.shape[1]
  k = 128
  nchunks = -(-e // (NW * k))
  nchunks += (-nchunks) % 16
  ew = nchunks * k

  e_pad = ew * NW
  n_pad = -(-n // (NS * 8)) * (NS * 8)
  if e_pad > e and n_pad == n:
    n_pad += NS * 8
  stripe = n_pad // NS

  ew_real = e // NW
  pad_w = ew - ew_real
  src_w = edge_index[0].reshape(NW, ew_real)
  dst_w = edge_index[1].reshape(NW, ew_real)
  if pad_w:
    ar = jnp.arange(pad_w, dtype=jnp.int32)
    wi = jnp.arange(NW, dtype=jnp.int32)[:, None]
    dummy_src = jnp.broadcast_to(n + (ar % (n_pad - n)), (NW, pad_w))
    dummy_dst = (ar[None, :] * 997 + wi * 131) % n
    src_w = jnp.concatenate([src_w, dummy_src], axis=1)
    dst_w = jnp.concatenate([dst_w, dummy_dst], axis=1)
  src_r = src_w.reshape(NC, NS, nchunks, k)
  dst_r = dst_w.reshape(NC, NS, nchunks, k)
  dst_g = edge_index[1].reshape(NC, NS, (e // NW) // 16, 16)
  zn = jnp.zeros((n_pad // 128, 128), jnp.float32)

  (cnt,) = _make_counts(n_pad, (e // NW) // 16)(dst_g, zn)
  cnt = cnt.reshape(NC, NS, n_pad)
  agg_a = _make_agg(n_pad, d, nchunks, k, False)
  x_p = jnp.pad(x, ((0, n_pad - n), (0, 0)))
  (sums1,) = agg_a(x_p, src_r, dst_r)
  bn = 128
  grid = (n_pad // bn,)
  row_spec = pl.BlockSpec((bn, d), lambda i: (i, 0))
  sum_spec = pl.BlockSpec((NC, bn, d), lambda i: (0, i, 0))
  cnt_spec = pl.BlockSpec((NC, NS, bn), lambda i: (0, 0, i))
  w_spec = pl.BlockSpec((d, d), lambda i: (0, 0))
  b_spec = pl.BlockSpec((1, d), lambda i: (0, 0))

  h1 = pl.pallas_call(
      _tc1_body,
      grid=grid,
      in_specs=[row_spec, sum_spec, cnt_spec, w_spec, w_spec],
      out_specs=row_spec,
      out_shape=jax.ShapeDtypeStruct((n_pad, d), jnp.float32),
  )(x_p, sums1, cnt, W_l1, W_r1)

  agg_b = _make_agg(n_pad, d, nchunks, k, False)
  (sums2,) = agg_b(h1, src_r, dst_r)

  out = pl.pallas_call(
      _tc2_body,
      grid=grid,
      in_specs=[row_spec, sum_spec, cnt_spec, w_spec, w_spec,
                w_spec, b_spec, w_spec, b_spec],
      out_specs=row_spec,
      out_shape=jax.ShapeDtypeStruct((n, d), jnp.float32),
  )(h1, sums2, cnt, W_l2, W_r2, W_p1, b_p1.reshape(1, d), W_p2,
    b_p2.reshape(1, d))
  return out
